# Initial kernel scaffold; baseline (speedup 1.0000x reference)
import jax
import jax.numpy as jnp
from jax.experimental import pallas as pl


def kernel(x, cw0, cb0, cw1, cb1, cw2, cb2, cw3, cb3, cw4, cb4, g0, b0, g1, b1, g2, b2, g3, b3, g4, b4, dw0, dg0, db0, dw1, dg1, db1, dw2, dg2, db2, dw3, dg3, db3, dw4, dg4, db4, dw5, dg5, db5, cew, ceg, ceb, Wq, bq, Wk, bk, Wv, bv, Wo, bo, sigma, gamma_p, rw1, rb1, rw2, rb2):
    raise NotImplementedError("write your pallas kernel here")


# speedup vs baseline: 17.2846x; 17.2846x over previous
import functools
import math

import jax
import jax.numpy as jnp
from jax.experimental import pallas as pl

_B, _N = 16, 1024
_KMIN, _KMAX = 5, 20
_HEADS = 4
_EPS = 1e-5


def _knn_body(x_ref, dist_ref, idx_ref):
    p = x_ref[0]
    cmask = jax.lax.broadcasted_iota(jnp.int32, (1, 4), 1) < 3
    p3 = jnp.where(cmask, p, 0.0)
    g = jnp.dot(p3, p3.T, preferred_element_type=jnp.float32)
    sq = jnp.sum(p3 * p3, axis=1, keepdims=True)
    d2 = sq + sq.T - 2.0 * g
    d = jnp.sqrt(jnp.clip(d2, 1e-12, None))
    iota = jax.lax.broadcasted_iota(jnp.int32, (_N, _N), 1)
    dcols, icols = [], []
    for _ in range(_KMAX):
        cur = jnp.min(d, axis=1, keepdims=True)
        am = jnp.min(jnp.where(d == cur, iota, _N), axis=1, keepdims=True)
        dcols.append(cur)
        icols.append(am)
        d = jnp.where(iota == am, jnp.inf, d)
    dist_ref[0] = jnp.concatenate(dcols, axis=1)
    idx_ref[0] = jnp.concatenate(icols, axis=1)


def _knn(xr):
    return pl.pallas_call(
        _knn_body,
        grid=(_B,),
        in_specs=[pl.BlockSpec((1, _N, 4), lambda b: (b, 0, 0))],
        out_specs=[pl.BlockSpec((1, _N, _KMAX), lambda b: (b, 0, 0)),
                   pl.BlockSpec((1, _N, _KMAX), lambda b: (b, 0, 0))],
        out_shape=[jax.ShapeDtypeStruct((_B, _N, _KMAX), jnp.float32),
                   jax.ShapeDtypeStruct((_B, _N, _KMAX), jnp.int32)],
    )(xr)


def _bn_rows(y, g, b, slope):
    m = jnp.mean(y, axis=0, keepdims=True)
    v = jnp.mean(y * y, axis=0, keepdims=True) - m * m
    z = (y - m) / jnp.sqrt(v + _EPS) * g + b
    if slope == 1.0:
        return z
    return jnp.where(z >= 0, z, slope * z)


def _mlp_body(x_ref, w0, b0r, w1, b1r, w2, b2r, w3, b3r,
              g0, be0, g1, be1, g2, be2, g3, be3, out_ref):
    h = jnp.dot(x_ref[...], w0[...].T, preferred_element_type=jnp.float32) + b0r[...]
    h = _bn_rows(h, g0[...], be0[...], 0.0)
    h = jnp.dot(h, w1[...].T, preferred_element_type=jnp.float32) + b1r[...]
    h = _bn_rows(h, g1[...], be1[...], 0.0)
    h = jnp.dot(h, w2[...].T, preferred_element_type=jnp.float32) + b2r[...]
    h = _bn_rows(h, g2[...], be2[...], 0.0)
    h = jnp.dot(h, w3[...].T, preferred_element_type=jnp.float32) + b3r[...]
    out_ref[...] = _bn_rows(h, g3[...], be3[...], 0.0)


def _mlp(xf, params):
    return pl.pallas_call(
        _mlp_body,
        out_shape=jax.ShapeDtypeStruct((_B * _N, 128), jnp.float32),
    )(xf, *params)


def _gather_conv_body(fea_ref, idx_ref, dist_ref, w_ref, out_ref, *, k):
    f = fea_ref[0]
    idx = idx_ref[0]
    iota = jax.lax.broadcasted_iota(jnp.int32, (_N, _N), 1)
    feat = jnp.full(f.shape, -jnp.inf, f.dtype)
    for j in range(k):
        oh = (idx[:, j:j + 1] == iota).astype(jnp.float32)
        gj = jnp.dot(oh, f, preferred_element_type=jnp.float32)
        feat = jnp.maximum(feat, gj)
    dmax = jnp.max(dist_ref[0], axis=1, keepdims=True)
    edge = jnp.concatenate([f, feat - f, dmax], axis=1)
    out_ref[0] = jnp.dot(edge, w_ref[...].T, preferred_element_type=jnp.float32)


def _gather_conv(fea, idx, dist, w, k):
    c = fea.shape[-1]
    co = w.shape[0]
    return pl.pallas_call(
        functools.partial(_gather_conv_body, k=k),
        grid=(_B,),
        in_specs=[pl.BlockSpec((1, _N, c), lambda b: (b, 0, 0)),
                  pl.BlockSpec((1, _N, k), lambda b: (b, 0, 0)),
                  pl.BlockSpec((1, _N, k), lambda b: (b, 0, 0)),
                  pl.BlockSpec(w.shape, lambda b: (0, 0))],
        out_specs=pl.BlockSpec((1, _N, co), lambda b: (b, 0, 0)),
        out_shape=jax.ShapeDtypeStruct((_B, _N, co), jnp.float32),
    )(fea, idx, dist, w)


def _gather_rcs_body(fea_ref, idx_ref, dist_ref, w_ref, rw_ref, rb_ref,
                     out_ref, rcs_ref, *, k):
    f = fea_ref[0]
    idx = idx_ref[0]
    iota = jax.lax.broadcasted_iota(jnp.int32, (_N, _N), 1)
    feat = jnp.full(f.shape, -jnp.inf, f.dtype)
    cols = [f[:, 3:4]]
    for j in range(k):
        oh = (idx[:, j:j + 1] == iota).astype(jnp.float32)
        gj = jnp.dot(oh, f, preferred_element_type=jnp.float32)
        feat = jnp.maximum(feat, gj)
        cols.append(gj[:, 3:4])
    rcs = jnp.concatenate(cols, axis=1)
    rcs_ref[0] = jnp.dot(rcs, rw_ref[...].T,
                         preferred_element_type=jnp.float32) + rb_ref[...]
    dmax = jnp.max(dist_ref[0], axis=1, keepdims=True)
    base = f[:, :3]
    edge = jnp.concatenate([base, feat[:, :3] - base, dmax], axis=1)
    out_ref[0] = jnp.dot(edge, w_ref[...].T, preferred_element_type=jnp.float32)


def _gather_rcs(fea, idx, dist, w, rw, rb, k):
    return pl.pallas_call(
        functools.partial(_gather_rcs_body, k=k),
        grid=(_B,),
        in_specs=[pl.BlockSpec((1, _N, 4), lambda b: (b, 0, 0)),
                  pl.BlockSpec((1, _N, k), lambda b: (b, 0, 0)),
                  pl.BlockSpec((1, _N, k), lambda b: (b, 0, 0)),
                  pl.BlockSpec(w.shape, lambda b: (0, 0)),
                  pl.BlockSpec(rw.shape, lambda b: (0, 0)),
                  pl.BlockSpec(rb.shape, lambda b: (0, 0))],
        out_specs=[pl.BlockSpec((1, _N, w.shape[0]), lambda b: (b, 0, 0)),
                   pl.BlockSpec((1, _N, rw.shape[0]), lambda b: (b, 0, 0))],
        out_shape=[jax.ShapeDtypeStruct((_B, _N, w.shape[0]), jnp.float32),
                   jax.ShapeDtypeStruct((_B, _N, rw.shape[0]), jnp.float32)],
    )(fea, idx, dist, w, rw, rb)


def _bn_act_body(y_ref, g_ref, b_ref, o_ref, *, slope):
    o_ref[...] = _bn_rows(y_ref[...], g_ref[...], b_ref[...], slope)


def _bn_act(y, g, b, slope):
    return pl.pallas_call(
        functools.partial(_bn_act_body, slope=slope),
        out_shape=jax.ShapeDtypeStruct(y.shape, jnp.float32),
    )(y, g, b)


def _mix_body(y5_ref, g5, b5, y6_ref, g6, b6, sig_ref, o_ref):
    z5 = _bn_rows(y5_ref[...], g5[...], b5[...], 0.2)
    z6 = _bn_rows(y6_ref[...], g6[...], b6[...], 0.2)
    s = sig_ref[0, 0]
    o_ref[...] = s * z5 + (1.0 - s) * z6


def _mix(y5, g5, b5, y6, g6, b6, sigma):
    return pl.pallas_call(
        _mix_body,
        out_shape=jax.ShapeDtypeStruct(y5.shape, jnp.float32),
    )(y5, g5, b5, y6, g6, b6, sigma)


def _mha_body(h_ref, x3_ref, wq, bq, wk, bk, wv, bv, wo, bo, out_ref):
    hq = h_ref[0]
    ctx = x3_ref[0]
    q = jnp.dot(hq, wq[...].T, preferred_element_type=jnp.float32) + bq[...]
    k_ = jnp.dot(ctx, wk[...].T, preferred_element_type=jnp.float32) + bk[...]
    v = jnp.dot(ctx, wv[...].T, preferred_element_type=jnp.float32) + bv[...]
    dh = 128 // _HEADS
    scale = 1.0 / math.sqrt(1.0 * dh)
    outs = []
    for hh in range(_HEADS):
        qh = q[:, hh * dh:(hh + 1) * dh]
        kh = k_[:, hh * dh:(hh + 1) * dh]
        vh = v[:, hh * dh:(hh + 1) * dh]
        s = jnp.dot(qh, kh.T, preferred_element_type=jnp.float32) * scale
        s = s - jnp.max(s, axis=1, keepdims=True)
        e = jnp.exp(s)
        p = e / jnp.sum(e, axis=1, keepdims=True)
        outs.append(jnp.dot(p, vh, preferred_element_type=jnp.float32))
    o = jnp.concatenate(outs, axis=1)
    out_ref[0] = jnp.dot(o, wo[...].T, preferred_element_type=jnp.float32) + bo[...]


def _mha(h, x3, wq, bq, wk, bk, wv, bv, wo, bo):
    wspec = [pl.BlockSpec(a.shape, lambda b: (0,) * a.ndim)
             for a in (wq, bq, wk, bk, wv, bv, wo, bo)]
    return pl.pallas_call(
        _mha_body,
        grid=(_B,),
        in_specs=[pl.BlockSpec((1, _N, 128), lambda b: (b, 0, 0)),
                  pl.BlockSpec((1, _N, 128), lambda b: (b, 0, 0))] + wspec,
        out_specs=pl.BlockSpec((1, _N, 128), lambda b: (b, 0, 0)),
        out_shape=jax.ShapeDtypeStruct((_B, _N, 128), jnp.float32),
    )(h, x3, wq, bq, wk, bk, wv, bv, wo, bo)


def _cew_body(h_ref, a_ref, wA, wB, ceg, ceb, w4, b4r, g4, be4, out_ref):
    y = (jnp.dot(h_ref[...], wA[...].T, preferred_element_type=jnp.float32)
         + jnp.dot(a_ref[...], wB[...].T, preferred_element_type=jnp.float32))
    x3e = _bn_rows(y, ceg[...], ceb[...], 0.0)
    y2 = jnp.dot(x3e, w4[...].T, preferred_element_type=jnp.float32) + b4r[...]
    out_ref[...] = _bn_rows(y2, g4[...], be4[...], 1.0)


def _cew(h, a, wA, wB, ceg, ceb, w4, b4r, g4, be4):
    return pl.pallas_call(
        _cew_body,
        out_shape=jax.ShapeDtypeStruct((_B * _N, 128), jnp.float32),
    )(h, a, wA, wB, ceg, ceb, w4, b4r, g4, be4)


def _fattn_body(f1_ref, f2_ref, o_ref, gam_ref, out_ref):
    f1 = f1_ref[0]
    f2 = f2_ref[0]
    ob = o_ref[0]
    s = jnp.dot(f1, f2.T, preferred_element_type=jnp.float32) / math.sqrt(32.0)
    s = s - jnp.max(s, axis=1, keepdims=True)
    e = jnp.exp(s)
    p = e / jnp.sum(e, axis=1, keepdims=True)
    enh = jnp.dot(p, ob, preferred_element_type=jnp.float32)
    out_ref[0] = ob + gam_ref[0, 0] * enh


def _fattn(f1, f2, o, gamma):
    return pl.pallas_call(
        _fattn_body,
        grid=(_B,),
        in_specs=[pl.BlockSpec((1, _N, 32), lambda b: (b, 0, 0)),
                  pl.BlockSpec((1, _N, 32), lambda b: (b, 0, 0)),
                  pl.BlockSpec((1, _N, 128), lambda b: (b, 0, 0)),
                  pl.BlockSpec((1, 1), lambda b: (0, 0))],
        out_specs=pl.BlockSpec((1, _N, 128), lambda b: (b, 0, 0)),
        out_shape=jax.ShapeDtypeStruct((_B, _N, 128), jnp.float32),
    )(f1, f2, o, gamma)


def kernel(x, cw0, cb0, cw1, cb1, cw2, cb2, cw3, cb3, cw4, cb4, g0, b0, g1, b1, g2, b2, g3, b3, g4, b4, dw0, dg0, db0, dw1, dg1, db1, dw2, dg2, db2, dw3, dg3, db3, dw4, dg4, db4, dw5, dg5, db5, cew, ceg, ceb, Wq, bq, Wk, bk, Wv, bv, Wo, bo, sigma, gamma_p, rw1, rb1, rw2, rb2):
    xr = x[:, 0]
    xf = xr.reshape(_B * _N, 4)

    r2 = lambda a: a.reshape(1, -1)

    dist20, idx20 = _knn(xr)
    dist5, idx5 = dist20[:, :, :_KMIN], idx20[:, :, :_KMIN]

    h = _mlp(xf, (cw0, r2(cb0), cw1, r2(cb1), cw2, r2(cb2), cw3, r2(cb3),
                  r2(g0), r2(b0), r2(g1), r2(b1), r2(g2), r2(b2), r2(g3), r2(b3)))

    y1, f1 = _gather_rcs(xr, idx5, dist5, dw0, rw1, r2(rb1), _KMIN)
    y2, f2 = _gather_rcs(xr, idx20, dist20, dw1, rw2, r2(rb2), _KMAX)
    z1 = _bn_act(y1.reshape(_B * _N, 64), r2(dg0), r2(db0), 0.2)
    z2 = _bn_act(y2.reshape(_B * _N, 64), r2(dg1), r2(db1), 0.2)
    xg1 = jnp.concatenate([z1.reshape(_B, _N, 64), xr[:, :, :3]], axis=2)
    xg2 = jnp.concatenate([z2.reshape(_B, _N, 64), xr[:, :, :3]], axis=2)

    y3 = _gather_conv(xg1, idx5, dist5, dw2, _KMIN)
    y4 = _gather_conv(xg2, idx20, dist20, dw3, _KMAX)
    z3 = _bn_act(y3.reshape(_B * _N, 64), r2(dg2), r2(db2), 0.2)
    z4 = _bn_act(y4.reshape(_B * _N, 64), r2(dg3), r2(db3), 0.2)
    xg3 = jnp.concatenate([z3.reshape(_B, _N, 64), xg1], axis=2)
    xg4 = jnp.concatenate([z4.reshape(_B, _N, 64), xg2], axis=2)

    y5 = _gather_conv(xg3, idx5, dist5, dw4, _KMIN)
    y6 = _gather_conv(xg4, idx20, dist20, dw5, _KMAX)
    x3 = _mix(y5.reshape(_B * _N, 128), r2(dg4), r2(db4),
              y6.reshape(_B * _N, 128), r2(dg5), r2(db5),
              sigma.reshape(1, 1))

    a = _mha(h.reshape(_B, _N, 128), x3.reshape(_B, _N, 128),
             Wq, r2(bq), Wk, r2(bk), Wv, r2(bv), Wo, r2(bo))

    outp = _cew(h, a.reshape(_B * _N, 128), cew[:, :128], cew[:, 128:],
                r2(ceg), r2(ceb), cw4, r2(cb4), r2(g4), r2(b4))

    res = _fattn(f1, f2, outp.reshape(_B, _N, 128), gamma_p.reshape(1, 1))
    return jnp.transpose(res, (0, 2, 1))[..., None]
